# Initial kernel scaffold; baseline (speedup 1.0000x reference)
import functools

import jax
import jax.numpy as jnp
from jax import lax
from jax.experimental import pallas as pl
from jax.experimental.pallas import tpu as pltpu
from jax.experimental.pallas import tpu_sc as plsc

_NC, _NS, _L = 2, 16, 16
_NW = _NC * _NS

_DEGW = 16


def _seg_sum_kernel(n, d, e):
    per_w = e // _NW
    ch = 80
    n_ch = per_w // ch
    rpt = n // _NS
    zrows = 125
    nz = rpt // zrows
    mesh = plsc.VectorSubcoreMesh(core_axis_name="c", subcore_axis_name="s")

    @functools.partial(
        pl.kernel,
        out_type=jax.ShapeDtypeStruct((_NC * n, d), jnp.float32),
        mesh=mesh,
        scratch_types=[
            pltpu.VMEM((ch,), jnp.int32),
            pltpu.VMEM((ch,), jnp.int32),
            pltpu.VMEM((ch, d), jnp.float32),
            pltpu.VMEM((zrows, d), jnp.float32),
            pltpu.VMEM_SHARED((n, d), jnp.float32),
            pltpu.SemaphoreType.DMA,
        ],
    )
    def seg(x_hbm, src_hbm, dst_hbm, out_hbm, sidx, didx, rows, zbuf, acc, sem):
        cid = lax.axis_index("c")
        sid = lax.axis_index("s")
        wid = cid * _NS + sid

        def _zr(r, _):
            def _zc(c, _):
                zbuf[r, pl.ds(c * _L, _L)] = jnp.zeros((_L,), jnp.float32)
                return 0
            return lax.fori_loop(0, d // _L, _zc, 0)
        lax.fori_loop(0, zrows, _zr, 0)

        row0 = sid * rpt

        def _zs(j, _):
            pltpu.sync_copy(zbuf, acc.at[pl.ds(row0 + j * zrows, zrows)])
            return 0
        lax.fori_loop(0, nz, _zs, 0)
        plsc.subcore_barrier()

        base = wid * per_w

        def _chunk(k, _):
            off = base + k * ch
            pltpu.sync_copy(src_hbm.at[pl.ds(off, ch)], sidx)
            pltpu.sync_copy(dst_hbm.at[pl.ds(off, ch)], didx)
            pltpu.async_copy(x_hbm.at[sidx], rows, sem).wait()
            pltpu.sync_copy(rows, acc.at[didx], add=True)
            return 0
        lax.fori_loop(0, n_ch, _chunk, 0)
        plsc.subcore_barrier()

        def _wb(j, _):
            r = row0 + j * zrows
            pltpu.sync_copy(acc.at[pl.ds(r, zrows)],
                            out_hbm.at[pl.ds(cid * n + r, zrows)])
            return 0
        lax.fori_loop(0, nz, _wb, 0)

    return seg


def _deg_kernel(n, e):
    per_w = e // _NW
    ch = 80
    n_ch = per_w // ch
    rpt = n // _NS
    zrows = 625
    nz = rpt // zrows
    mesh = plsc.VectorSubcoreMesh(core_axis_name="c", subcore_axis_name="s")

    @functools.partial(
        pl.kernel,
        out_type=jax.ShapeDtypeStruct((_NC * n, _DEGW), jnp.float32),
        mesh=mesh,
        scratch_types=[
            pltpu.VMEM((ch,), jnp.int32),
            pltpu.VMEM((ch, _DEGW), jnp.float32),
            pltpu.VMEM((zrows, _DEGW), jnp.float32),
            pltpu.VMEM_SHARED((n, _DEGW), jnp.float32),
        ],
    )
    def deg(dst_hbm, out_hbm, didx, ones, zbuf, acc):
        cid = lax.axis_index("c")
        sid = lax.axis_index("s")
        wid = cid * _NS + sid

        def _init(r, _):
            ones[r, :] = jnp.full((_DEGW,), 1.0, jnp.float32)
            return 0
        lax.fori_loop(0, ch, _init, 0)

        def _zr(r, _):
            zbuf[r, :] = jnp.zeros((_DEGW,), jnp.float32)
            return 0
        lax.fori_loop(0, zrows, _zr, 0)

        row0 = sid * rpt

        def _zs(j, _):
            pltpu.sync_copy(zbuf, acc.at[pl.ds(row0 + j * zrows, zrows)])
            return 0
        lax.fori_loop(0, nz, _zs, 0)
        plsc.subcore_barrier()

        base = wid * per_w

        def _chunk(k, _):
            pltpu.sync_copy(dst_hbm.at[pl.ds(base + k * ch, ch)], didx)
            pltpu.sync_copy(ones, acc.at[didx], add=True)
            return 0
        lax.fori_loop(0, n_ch, _chunk, 0)
        plsc.subcore_barrier()

        def _wb(j, _):
            r = row0 + j * zrows
            pltpu.sync_copy(acc.at[pl.ds(r, zrows)],
                            out_hbm.at[pl.ds(cid * n + r, zrows)])
            return 0
        lax.fori_loop(0, nz, _wb, 0)

    return deg


def _dense_layer(n, d, h, br):
    grid = (n // br,)

    def body(x_ref, p_ref, dg_ref, ws_ref, wn_ref, b_ref, o_ref):
        p = p_ref[0] + p_ref[1]
        deg = dg_ref[0, :, :1] + dg_ref[1, :, :1]
        hn = p / jnp.maximum(deg, 1.0)
        y = (jnp.dot(x_ref[...], ws_ref[...], preferred_element_type=jnp.float32)
             + jnp.dot(hn, wn_ref[...], preferred_element_type=jnp.float32)
             + b_ref[...])
        o_ref[...] = jnp.maximum(y, 0.0)

    return pl.pallas_call(
        body,
        grid=grid,
        in_specs=[
            pl.BlockSpec((br, d), lambda i: (i, 0)),
            pl.BlockSpec((_NC, br, d), lambda i: (0, i, 0)),
            pl.BlockSpec((_NC, br, _DEGW), lambda i: (0, i, 0)),
            pl.BlockSpec((d, h), lambda i: (0, 0)),
            pl.BlockSpec((d, h), lambda i: (0, 0)),
            pl.BlockSpec((1, h), lambda i: (0, 0)),
        ],
        out_specs=pl.BlockSpec((br, h), lambda i: (i, 0)),
        out_shape=jax.ShapeDtypeStruct((n, h), jnp.float32),
    )


def _final_layer(n, d, h, fc2, c, br):
    nb = n // br

    def body(x_ref, p_ref, dg_ref, ws_ref, wn_ref, b_ref,
             wf1_ref, bf1_ref, wf2_ref, bf2_ref, o_ref, acc_ref):
        i = pl.program_id(0)

        @pl.when(i == 0)
        def _():
            acc_ref[...] = jnp.zeros_like(acc_ref)

        p = p_ref[0] + p_ref[1]
        deg = dg_ref[0, :, :1] + dg_ref[1, :, :1]
        hn = p / jnp.maximum(deg, 1.0)
        y = (jnp.dot(x_ref[...], ws_ref[...], preferred_element_type=jnp.float32)
             + jnp.dot(hn, wn_ref[...], preferred_element_type=jnp.float32)
             + b_ref[...])
        y = jnp.maximum(y, 0.0)
        acc_ref[...] += jnp.sum(y, axis=0, keepdims=True)

        @pl.when(i == nb - 1)
        def _():
            hg = acc_ref[...] / float(n)
            t1 = jnp.dot(hg, wf1_ref[...], preferred_element_type=jnp.float32)
            t1 = jnp.maximum(t1 + bf1_ref[...], 0.0)
            t2 = jnp.dot(t1, wf2_ref[...], preferred_element_type=jnp.float32)
            t2 = t2 + bf2_ref[...]
            m = jnp.max(t2)
            lse = m + jnp.log(jnp.sum(jnp.exp(t2 - m)))
            o_ref[...] = t2 - lse

    return pl.pallas_call(
        body,
        grid=(nb,),
        in_specs=[
            pl.BlockSpec((br, d), lambda i: (i, 0)),
            pl.BlockSpec((_NC, br, d), lambda i: (0, i, 0)),
            pl.BlockSpec((_NC, br, _DEGW), lambda i: (0, i, 0)),
            pl.BlockSpec((d, h), lambda i: (0, 0)),
            pl.BlockSpec((d, h), lambda i: (0, 0)),
            pl.BlockSpec((1, h), lambda i: (0, 0)),
            pl.BlockSpec((h, fc2), lambda i: (0, 0)),
            pl.BlockSpec((1, fc2), lambda i: (0, 0)),
            pl.BlockSpec((fc2, c), lambda i: (0, 0)),
            pl.BlockSpec((1, c), lambda i: (0, 0)),
        ],
        out_specs=pl.BlockSpec((1, c), lambda i: (0, 0)),
        out_shape=jax.ShapeDtypeStruct((1, c), jnp.float32),
        scratch_shapes=[pltpu.VMEM((1, h), jnp.float32)],
    )


def kernel(x, edge_index, W1_self, W1_neigh, b1, W2_self, W2_neigh, b2,
           W3_self, W3_neigh, b3, Wfc1, bfc1, Wfc2, bfc2):
    n, d = x.shape
    e = edge_index.shape[1]
    h1 = W1_self.shape[1]
    h2 = W2_self.shape[1]
    fc1 = W3_self.shape[1]
    fc2 = Wfc1.shape[1]
    c = Wfc2.shape[1]
    br = 2000

    src = edge_index[0]
    dst = edge_index[1]

    seg = _seg_sum_kernel(n, d, e)
    degk = _deg_kernel(n, e)
    dense1 = _dense_layer(n, d, h1, br)
    dense2 = _dense_layer(n, h1, h2, br)
    dense3 = _final_layer(n, h2, fc1, fc2, c, br)

    degw = degk(dst).reshape(_NC, n, _DEGW)

    p1 = seg(x, src, dst).reshape(_NC, n, d)
    hh1 = dense1(x, p1, degw, W1_self, W1_neigh, b1.reshape(1, h1))
    p2 = seg(hh1, src, dst).reshape(_NC, n, d)
    hh2 = dense2(hh1, p2, degw, W2_self, W2_neigh, b2.reshape(1, h2))
    p3 = seg(hh2, src, dst).reshape(_NC, n, d)
    out = dense3(hh2, p3, degw, W3_self, W3_neigh, b3.reshape(1, fc1),
                 Wfc1, bfc1.reshape(1, fc2), Wfc2, bfc2.reshape(1, c))
    return out


# speedup vs baseline: 4.4368x; 4.4368x over previous
import functools

import jax
import jax.numpy as jnp
from jax import lax
from jax.experimental import pallas as pl
from jax.experimental.pallas import tpu as pltpu
from jax.experimental.pallas import tpu_sc as plsc

_NC, _NS, _L = 2, 16, 16
_NW = _NC * _NS


def _seg_sum_kernel(n, d, e, with_gather):
    per_w = e // _NW
    ch = 80
    n_ch = per_w // ch
    rpt = (n // _NS) // 8 * 8
    last_extra = n - _NS * rpt
    zrows = 104
    nz = rpt // zrows
    mesh = plsc.VectorSubcoreMesh(core_axis_name="c", subcore_axis_name="s")

    scratch = [
        pltpu.VMEM((ch,), jnp.int32),
        pltpu.VMEM((ch,), jnp.int32),
        pltpu.VMEM((ch, d), jnp.float32),
        pltpu.VMEM((zrows, d), jnp.float32),
        pltpu.VMEM_SHARED((n, d), jnp.float32),
        pltpu.SemaphoreType.DMA,
    ]

    @functools.partial(
        pl.kernel,
        out_type=jax.ShapeDtypeStruct((_NC * n, d), jnp.float32),
        mesh=mesh,
        scratch_types=scratch,
    )
    def seg(x_hbm, src_hbm, dst_hbm, out_hbm, sidx, didx, rows, zbuf, acc, sem):
        cid = lax.axis_index("c")
        sid = lax.axis_index("s")
        wid = cid * _NS + sid

        def _zr(r, _):
            def _zc(c, _):
                zbuf[r, pl.ds(c * _L, _L)] = jnp.zeros((_L,), jnp.float32)
                return 0
            return lax.fori_loop(0, d // _L, _zc, 0)
        lax.fori_loop(0, zrows, _zr, 0)

        if not with_gather:
            def _or(r, _):
                def _oc(c, _):
                    rows[r, pl.ds(c * _L, _L)] = jnp.full((_L,), 1.0, jnp.float32)
                    return 0
                return lax.fori_loop(0, d // _L, _oc, 0)
            lax.fori_loop(0, ch, _or, 0)

        row0 = pl.multiple_of(sid * rpt, 8)
        for j in range(nz):
            pltpu.sync_copy(zbuf, acc.at[pl.ds(row0 + j * zrows, zrows)])

        @pl.when(sid == _NS - 1)
        def _():
            pltpu.sync_copy(zbuf.at[pl.ds(0, last_extra)],
                            acc.at[pl.ds(_NS * rpt, last_extra)])
        plsc.subcore_barrier()

        base = wid * per_w

        def _chunk(k, _):
            off = base + k * ch
            pltpu.sync_copy(dst_hbm.at[pl.ds(off, ch)], didx)
            if with_gather:
                pltpu.sync_copy(src_hbm.at[pl.ds(off, ch)], sidx)
                pltpu.async_copy(x_hbm.at[sidx], rows, sem).wait()
            pltpu.sync_copy(rows, acc.at[didx], add=True)
            return 0
        lax.fori_loop(0, n_ch, _chunk, 0)
        plsc.subcore_barrier()

        out0 = pl.multiple_of(cid * n + row0, 8)
        pltpu.sync_copy(acc.at[pl.ds(row0, rpt)], out_hbm.at[pl.ds(out0, rpt)])

        @pl.when(sid == _NS - 1)
        def _():
            pltpu.sync_copy(acc.at[pl.ds(_NS * rpt, last_extra)],
                            out_hbm.at[pl.ds(cid * n + _NS * rpt, last_extra)])

    return seg


def _dense_layer(n, d, h, br):
    grid = (n // br,)

    def body(x_ref, p_ref, dg_ref, ws_ref, wn_ref, b_ref, o_ref):
        p = p_ref[0] + p_ref[1]
        deg = dg_ref[0] + dg_ref[1]
        hn = p / jnp.maximum(deg, 1.0)
        y = (jnp.dot(x_ref[...], ws_ref[...], preferred_element_type=jnp.float32)
             + jnp.dot(hn, wn_ref[...], preferred_element_type=jnp.float32)
             + b_ref[...])
        o_ref[...] = jnp.maximum(y, 0.0)

    return pl.pallas_call(
        body,
        grid=grid,
        in_specs=[
            pl.BlockSpec((br, d), lambda i: (i, 0)),
            pl.BlockSpec((_NC, br, d), lambda i: (0, i, 0)),
            pl.BlockSpec((_NC, br, d), lambda i: (0, i, 0)),
            pl.BlockSpec((d, h), lambda i: (0, 0)),
            pl.BlockSpec((d, h), lambda i: (0, 0)),
            pl.BlockSpec((1, h), lambda i: (0, 0)),
        ],
        out_specs=pl.BlockSpec((br, h), lambda i: (i, 0)),
        out_shape=jax.ShapeDtypeStruct((n, h), jnp.float32),
    )


def _final_layer(n, d, h, fc2, c, br):
    nb = n // br

    def body(x_ref, p_ref, dg_ref, ws_ref, wn_ref, b_ref,
             wf1_ref, bf1_ref, wf2_ref, bf2_ref, o_ref, acc_ref):
        i = pl.program_id(0)

        @pl.when(i == 0)
        def _():
            acc_ref[...] = jnp.zeros_like(acc_ref)

        p = p_ref[0] + p_ref[1]
        deg = dg_ref[0] + dg_ref[1]
        hn = p / jnp.maximum(deg, 1.0)
        y = (jnp.dot(x_ref[...], ws_ref[...], preferred_element_type=jnp.float32)
             + jnp.dot(hn, wn_ref[...], preferred_element_type=jnp.float32)
             + b_ref[...])
        y = jnp.maximum(y, 0.0)
        acc_ref[...] += jnp.sum(y, axis=0, keepdims=True)

        @pl.when(i == nb - 1)
        def _():
            hg = acc_ref[...] / float(n)
            t1 = jnp.dot(hg, wf1_ref[...], preferred_element_type=jnp.float32)
            t1 = jnp.maximum(t1 + bf1_ref[...], 0.0)
            t2 = jnp.dot(t1, wf2_ref[...], preferred_element_type=jnp.float32)
            t2 = t2 + bf2_ref[...]
            m = jnp.max(t2)
            lse = m + jnp.log(jnp.sum(jnp.exp(t2 - m)))
            o_ref[...] = t2 - lse

    return pl.pallas_call(
        body,
        grid=(nb,),
        in_specs=[
            pl.BlockSpec((br, d), lambda i: (i, 0)),
            pl.BlockSpec((_NC, br, d), lambda i: (0, i, 0)),
            pl.BlockSpec((_NC, br, d), lambda i: (0, i, 0)),
            pl.BlockSpec((d, h), lambda i: (0, 0)),
            pl.BlockSpec((d, h), lambda i: (0, 0)),
            pl.BlockSpec((1, h), lambda i: (0, 0)),
            pl.BlockSpec((h, fc2), lambda i: (0, 0)),
            pl.BlockSpec((1, fc2), lambda i: (0, 0)),
            pl.BlockSpec((fc2, c), lambda i: (0, 0)),
            pl.BlockSpec((1, c), lambda i: (0, 0)),
        ],
        out_specs=pl.BlockSpec((1, c), lambda i: (0, 0)),
        out_shape=jax.ShapeDtypeStruct((1, c), jnp.float32),
        scratch_shapes=[pltpu.VMEM((1, h), jnp.float32)],
    )


def kernel(x, edge_index, W1_self, W1_neigh, b1, W2_self, W2_neigh, b2,
           W3_self, W3_neigh, b3, Wfc1, bfc1, Wfc2, bfc2):
    n, d = x.shape
    e = edge_index.shape[1]
    h1 = W1_self.shape[1]
    h2 = W2_self.shape[1]
    fc1 = W3_self.shape[1]
    fc2 = Wfc1.shape[1]
    c = Wfc2.shape[1]
    br = 2000

    src = edge_index[0]
    dst = edge_index[1]

    seg = _seg_sum_kernel(n, d, e, with_gather=True)
    degk = _seg_sum_kernel(n, d, e, with_gather=False)
    dense1 = _dense_layer(n, d, h1, br)
    dense2 = _dense_layer(n, h1, h2, br)
    dense3 = _final_layer(n, h2, fc1, fc2, c, br)

    degw = degk(x, src, dst).reshape(_NC, n, d)

    p1 = seg(x, src, dst).reshape(_NC, n, d)
    hh1 = dense1(x, p1, degw, W1_self, W1_neigh, b1.reshape(1, h1))
    p2 = seg(hh1, src, dst).reshape(_NC, n, d)
    hh2 = dense2(hh1, p2, degw, W2_self, W2_neigh, b2.reshape(1, h2))
    p3 = seg(hh2, src, dst).reshape(_NC, n, d)
    out = dense3(hh2, p3, degw, W3_self, W3_neigh, b3.reshape(1, fc1),
                 Wfc1, bfc1.reshape(1, fc2), Wfc2, bfc2.reshape(1, c))
    return out


# speedup vs baseline: 7.4548x; 1.6802x over previous
import functools

import jax
import jax.numpy as jnp
from jax import lax
from jax.experimental import pallas as pl
from jax.experimental.pallas import tpu as pltpu
from jax.experimental.pallas import tpu_sc as plsc

_NC, _NS, _L = 2, 16, 16
_NW = _NC * _NS


def _seg_sum_kernel(n, d, e, with_gather):
    per_w = e // _NW
    ch = 80
    n_ch = per_w // ch
    NB = 2
    n_grp = n_ch // NB
    n_tail = n_ch - n_grp * NB
    rpt = (n // _NS) // 8 * 8
    last_extra = n - _NS * rpt
    nz = rpt // ch
    zrem = rpt - nz * ch
    mesh = plsc.VectorSubcoreMesh(core_axis_name="c", subcore_axis_name="s")

    scratch = [
        pltpu.VMEM((NB, ch), jnp.int32),
        pltpu.VMEM((n_ch, ch), jnp.int32),
        pltpu.VMEM((NB, ch, d), jnp.float32),
        pltpu.VMEM_SHARED((n, d), jnp.float32),
    ] + [pltpu.SemaphoreType.DMA] * (2 * NB)

    @functools.partial(
        pl.kernel,
        out_type=jax.ShapeDtypeStruct((_NC * n, d), jnp.float32),
        mesh=mesh,
        scratch_types=scratch,
    )
    def seg(x_hbm, src_hbm, dst_hbm, out_hbm, sidx, didx, rows, acc,
            *sems):
        gsem, isem = sems[:NB], sems[NB:]
        cid = lax.axis_index("c")
        sid = lax.axis_index("s")
        wid = cid * _NS + sid
        base = wid * per_w

        def _wait_idx(sem):
            pltpu.make_async_copy(dst_hbm.at[pl.ds(0, ch)], sidx.at[0],
                                  sem).wait()

        def _wait_gather(b):
            pltpu.make_async_copy(x_hbm.at[sidx.at[b]], rows.at[b],
                                  gsem[b]).wait()

        def _idx(t, _):
            for b in range(NB):
                k = t * NB + b

                @pl.when(t > 0)
                def _():
                    _wait_idx(isem[b])
                pltpu.async_copy(dst_hbm.at[pl.ds(base + k * ch, ch)],
                                 didx.at[k], isem[b])
            return 0
        lax.fori_loop(0, n_ch // NB, _idx, 0)
        for r in range(n_tail):
            k = (n_ch // NB) * NB + r
            _wait_idx(isem[r])
            pltpu.async_copy(dst_hbm.at[pl.ds(base + k * ch, ch)],
                             didx.at[k], isem[r])

        def _zr(r, _):
            def _zc(c, _):
                rows[0, r, pl.ds(c * _L, _L)] = jnp.zeros((_L,), jnp.float32)
                return 0
            return lax.fori_loop(0, d // _L, _zc, 0)
        lax.fori_loop(0, ch, _zr, 0)

        for b in range(NB):
            _wait_idx(isem[b])

        row0 = pl.multiple_of(sid * rpt, 8)
        for j in range(nz):
            pltpu.sync_copy(rows.at[0], acc.at[pl.ds(row0 + j * ch, ch)])
        pltpu.sync_copy(rows.at[0, pl.ds(0, zrem)],
                        acc.at[pl.ds(row0 + nz * ch, zrem)])

        @pl.when(sid == _NS - 1)
        def _():
            pltpu.sync_copy(rows.at[0, pl.ds(0, last_extra)],
                            acc.at[pl.ds(_NS * rpt, last_extra)])

        if with_gather:
            for b in range(NB):
                pltpu.async_copy(src_hbm.at[pl.ds(base + b * ch, ch)],
                                 sidx.at[b], isem[b])
            _wait_idx(isem[0])
            pltpu.async_copy(x_hbm.at[sidx.at[0]], rows.at[0], gsem[0])
        else:
            def _or(r, _):
                def _oc(c, _):
                    rows[0, r, pl.ds(c * _L, _L)] = jnp.full((_L,), 1.0,
                                                             jnp.float32)
                    return 0
                return lax.fori_loop(0, d // _L, _oc, 0)
            lax.fori_loop(0, ch, _or, 0)
        plsc.subcore_barrier()

        if with_gather:
            def _grp(t, _):
                for b in range(NB):
                    c = t * NB + b
                    nb = 1 - b

                    @pl.when(c + 1 < n_ch)
                    def _():
                        _wait_idx(isem[nb])
                        pltpu.async_copy(x_hbm.at[sidx.at[nb]], rows.at[nb],
                                         gsem[nb])
                    _wait_gather(b)
                    pltpu.sync_copy(rows.at[b], acc.at[didx.at[c]], add=True)

                    @pl.when(c + 2 < n_ch)
                    def _():
                        pltpu.async_copy(
                            src_hbm.at[pl.ds(base + (c + 2) * ch, ch)],
                            sidx.at[b], isem[b])
                return 0
            lax.fori_loop(0, n_grp, _grp, 0)
            for r in range(n_tail):
                c = n_grp * NB + r
                _wait_gather(r)
                pltpu.sync_copy(rows.at[r], acc.at[didx.at[c]], add=True)
        else:
            for b in range(NB):
                pltpu.async_copy(rows.at[0], acc.at[didx.at[b]], gsem[b],
                                 add=True)

            def _grp(t, _):
                for b in range(NB):
                    c = (t + 1) * NB + b
                    pltpu.make_async_copy(rows.at[0], acc.at[didx.at[0]],
                                          gsem[b]).wait()

                    @pl.when(c < n_ch)
                    def _():
                        pltpu.async_copy(rows.at[0], acc.at[didx.at[c]],
                                         gsem[b], add=True)
                return 0
            lax.fori_loop(0, (n_ch + NB - 1) // NB - 1, _grp, 0)
            rem = n_ch - ((n_ch + NB - 1) // NB - 1) * NB
            for b in range(rem):
                pltpu.make_async_copy(rows.at[0], acc.at[didx.at[0]],
                                      gsem[b]).wait()
        plsc.subcore_barrier()

        out0 = pl.multiple_of(cid * n + row0, 8)
        pltpu.sync_copy(acc.at[pl.ds(row0, rpt)], out_hbm.at[pl.ds(out0, rpt)])

        @pl.when(sid == _NS - 1)
        def _():
            pltpu.sync_copy(acc.at[pl.ds(_NS * rpt, last_extra)],
                            out_hbm.at[pl.ds(cid * n + _NS * rpt, last_extra)])

    return seg


def _dense_layer(n, d, h, br):
    grid = (n // br,)

    def body(x_ref, p_ref, dg_ref, ws_ref, wn_ref, b_ref, o_ref):
        p = p_ref[0] + p_ref[1]
        deg = dg_ref[0] + dg_ref[1]
        hn = p / jnp.maximum(deg, 1.0)
        y = (jnp.dot(x_ref[...], ws_ref[...], preferred_element_type=jnp.float32)
             + jnp.dot(hn, wn_ref[...], preferred_element_type=jnp.float32)
             + b_ref[...])
        o_ref[...] = jnp.maximum(y, 0.0)

    return pl.pallas_call(
        body,
        grid=grid,
        in_specs=[
            pl.BlockSpec((br, d), lambda i: (i, 0)),
            pl.BlockSpec((_NC, br, d), lambda i: (0, i, 0)),
            pl.BlockSpec((_NC, br, d), lambda i: (0, i, 0)),
            pl.BlockSpec((d, h), lambda i: (0, 0)),
            pl.BlockSpec((d, h), lambda i: (0, 0)),
            pl.BlockSpec((1, h), lambda i: (0, 0)),
        ],
        out_specs=pl.BlockSpec((br, h), lambda i: (i, 0)),
        out_shape=jax.ShapeDtypeStruct((n, h), jnp.float32),
    )


def _final_layer(n, d, h, fc2, c, br):
    nb = n // br

    def body(x_ref, p_ref, dg_ref, ws_ref, wn_ref, b_ref,
             wf1_ref, bf1_ref, wf2_ref, bf2_ref, o_ref, acc_ref):
        i = pl.program_id(0)

        @pl.when(i == 0)
        def _():
            acc_ref[...] = jnp.zeros_like(acc_ref)

        p = p_ref[0] + p_ref[1]
        deg = dg_ref[0] + dg_ref[1]
        hn = p / jnp.maximum(deg, 1.0)
        y = (jnp.dot(x_ref[...], ws_ref[...], preferred_element_type=jnp.float32)
             + jnp.dot(hn, wn_ref[...], preferred_element_type=jnp.float32)
             + b_ref[...])
        y = jnp.maximum(y, 0.0)
        acc_ref[...] += jnp.sum(y, axis=0, keepdims=True)

        @pl.when(i == nb - 1)
        def _():
            hg = acc_ref[...] / float(n)
            t1 = jnp.dot(hg, wf1_ref[...], preferred_element_type=jnp.float32)
            t1 = jnp.maximum(t1 + bf1_ref[...], 0.0)
            t2 = jnp.dot(t1, wf2_ref[...], preferred_element_type=jnp.float32)
            t2 = t2 + bf2_ref[...]
            m = jnp.max(t2)
            lse = m + jnp.log(jnp.sum(jnp.exp(t2 - m)))
            o_ref[...] = t2 - lse

    return pl.pallas_call(
        body,
        grid=(nb,),
        in_specs=[
            pl.BlockSpec((br, d), lambda i: (i, 0)),
            pl.BlockSpec((_NC, br, d), lambda i: (0, i, 0)),
            pl.BlockSpec((_NC, br, d), lambda i: (0, i, 0)),
            pl.BlockSpec((d, h), lambda i: (0, 0)),
            pl.BlockSpec((d, h), lambda i: (0, 0)),
            pl.BlockSpec((1, h), lambda i: (0, 0)),
            pl.BlockSpec((h, fc2), lambda i: (0, 0)),
            pl.BlockSpec((1, fc2), lambda i: (0, 0)),
            pl.BlockSpec((fc2, c), lambda i: (0, 0)),
            pl.BlockSpec((1, c), lambda i: (0, 0)),
        ],
        out_specs=pl.BlockSpec((1, c), lambda i: (0, 0)),
        out_shape=jax.ShapeDtypeStruct((1, c), jnp.float32),
        scratch_shapes=[pltpu.VMEM((1, h), jnp.float32)],
    )


def kernel(x, edge_index, W1_self, W1_neigh, b1, W2_self, W2_neigh, b2,
           W3_self, W3_neigh, b3, Wfc1, bfc1, Wfc2, bfc2):
    n, d = x.shape
    e = edge_index.shape[1]
    h1 = W1_self.shape[1]
    h2 = W2_self.shape[1]
    fc1 = W3_self.shape[1]
    fc2 = Wfc1.shape[1]
    c = Wfc2.shape[1]
    br = 2000

    src3 = edge_index[0]
    dst3 = edge_index[1]

    seg = _seg_sum_kernel(n, d, e, with_gather=True)
    degk = _seg_sum_kernel(n, d, e, with_gather=False)
    dense1 = _dense_layer(n, d, h1, br)
    dense2 = _dense_layer(n, h1, h2, br)
    dense3 = _final_layer(n, h2, fc1, fc2, c, br)

    degw = degk(x, src3, dst3).reshape(_NC, n, d)

    p1 = seg(x, src3, dst3).reshape(_NC, n, d)
    hh1 = dense1(x, p1, degw, W1_self, W1_neigh, b1.reshape(1, h1))
    p2 = seg(hh1, src3, dst3).reshape(_NC, n, d)
    hh2 = dense2(hh1, p2, degw, W2_self, W2_neigh, b2.reshape(1, h2))
    p3 = seg(hh2, src3, dst3).reshape(_NC, n, d)
    out = dense3(hh2, p3, degw, W3_self, W3_neigh, b3.reshape(1, fc1),
                 Wfc1, bfc1.reshape(1, fc2), Wfc2, bfc2.reshape(1, c))
    return out


# speedup vs baseline: 8.3079x; 1.1144x over previous
import functools

import jax
import jax.numpy as jnp
from jax import lax
from jax.experimental import pallas as pl
from jax.experimental.pallas import tpu as pltpu
from jax.experimental.pallas import tpu_sc as plsc

_NC, _NS, _L = 2, 16, 16
_NW = _NC * _NS


def _seg_sum_kernel(n, d, e, with_gather):
    per_w = e // _NW
    ch = 80
    n_ch = per_w // ch
    NB = 2
    n_grp = n_ch // NB
    n_tail = n_ch - n_grp * NB
    rpt = (n // _NS) // 8 * 8
    last_extra = n - _NS * rpt
    nz = rpt // ch
    zrem = rpt - nz * ch
    mesh = plsc.VectorSubcoreMesh(core_axis_name="c", subcore_axis_name="s")

    scratch = [
        pltpu.VMEM((NB, ch), jnp.int32),
        pltpu.VMEM((n_ch, ch), jnp.int32),
        pltpu.VMEM((NB, ch, d), jnp.float32),
        pltpu.VMEM_SHARED((n, d), jnp.float32),
    ] + [pltpu.SemaphoreType.DMA] * (3 * NB)

    @functools.partial(
        pl.kernel,
        out_type=jax.ShapeDtypeStruct((_NC * n, d), jnp.float32),
        mesh=mesh,
        scratch_types=scratch,
    )
    def seg(x_hbm, src_hbm, dst_hbm, out_hbm, sidx, didx, rows, acc,
            *sems):
        gsem, isem, ssem = sems[:NB], sems[NB:2 * NB], sems[2 * NB:]
        cid = lax.axis_index("c")
        sid = lax.axis_index("s")
        wid = cid * _NS + sid
        base = wid * per_w

        def _wait_idx(sem):
            pltpu.make_async_copy(dst_hbm.at[pl.ds(0, ch)], sidx.at[0],
                                  sem).wait()

        def _wait_gather(b):
            pltpu.make_async_copy(x_hbm.at[sidx.at[b]], rows.at[b],
                                  gsem[b]).wait()

        def _idx(t, _):
            for b in range(NB):
                k = t * NB + b

                @pl.when(t > 0)
                def _():
                    _wait_idx(isem[b])
                pltpu.async_copy(dst_hbm.at[pl.ds(base + k * ch, ch)],
                                 didx.at[k], isem[b])
            return 0
        lax.fori_loop(0, n_ch // NB, _idx, 0)
        for r in range(n_tail):
            k = (n_ch // NB) * NB + r
            _wait_idx(isem[r])
            pltpu.async_copy(dst_hbm.at[pl.ds(base + k * ch, ch)],
                             didx.at[k], isem[r])

        def _zr(r, _):
            def _zc(c, _):
                rows[0, r, pl.ds(c * _L, _L)] = jnp.zeros((_L,), jnp.float32)
                return 0
            return lax.fori_loop(0, d // _L, _zc, 0)
        lax.fori_loop(0, ch, _zr, 0)

        for b in range(NB):
            _wait_idx(isem[b])

        row0 = pl.multiple_of(sid * rpt, 8)
        for j in range(nz):
            pltpu.sync_copy(rows.at[0], acc.at[pl.ds(row0 + j * ch, ch)])
        pltpu.sync_copy(rows.at[0, pl.ds(0, zrem)],
                        acc.at[pl.ds(row0 + nz * ch, zrem)])

        @pl.when(sid == _NS - 1)
        def _():
            pltpu.sync_copy(rows.at[0, pl.ds(0, last_extra)],
                            acc.at[pl.ds(_NS * rpt, last_extra)])

        if with_gather:
            for b in range(NB):
                pltpu.async_copy(src_hbm.at[pl.ds(base + b * ch, ch)],
                                 sidx.at[b], isem[b])
            _wait_idx(isem[0])
            pltpu.async_copy(x_hbm.at[sidx.at[0]], rows.at[0], gsem[0])
        else:
            def _or(r, _):
                def _oc(c, _):
                    rows[0, r, pl.ds(c * _L, _L)] = jnp.full((_L,), 1.0,
                                                             jnp.float32)
                    return 0
                return lax.fori_loop(0, d // _L, _oc, 0)
            lax.fori_loop(0, ch, _or, 0)
        plsc.subcore_barrier()

        if with_gather:
            def _wait_scat(b):
                pltpu.make_async_copy(rows.at[b], acc.at[didx.at[0]],
                                      ssem[b]).wait()

            def _step(c, b, first):
                nb = 1 - b

                @pl.when(c + 1 < n_ch)
                def _():
                    _wait_idx(isem[nb])
                    if not first:
                        _wait_scat(nb)
                    pltpu.async_copy(x_hbm.at[sidx.at[nb]], rows.at[nb],
                                     gsem[nb])
                _wait_gather(b)
                pltpu.async_copy(rows.at[b], acc.at[didx.at[c]], ssem[b],
                                 add=True)

                @pl.when(c + 2 < n_ch)
                def _():
                    pltpu.async_copy(
                        src_hbm.at[pl.ds(base + (c + 2) * ch, ch)],
                        sidx.at[b], isem[b])

            _step(0, 0, True)

            def _grp(t, _):
                for b in range(NB):
                    _step(1 + t * NB + b, 1 - b if NB == 2 else b, False)
                return 0
            lax.fori_loop(0, (n_ch - 1) // NB, _grp, 0)
            _wait_scat((n_ch - 2) % NB)
            _wait_scat((n_ch - 1) % NB)
        else:
            dsems = list(gsem) + list(isem)
            nd = len(dsems)
            for b in range(nd):
                pltpu.async_copy(rows.at[0], acc.at[didx.at[b]], dsems[b],
                                 add=True)

            def _grp(t, _):
                for b in range(nd):
                    c = (t + 1) * nd + b
                    pltpu.make_async_copy(rows.at[0], acc.at[didx.at[0]],
                                          dsems[b]).wait()

                    @pl.when(c < n_ch)
                    def _():
                        pltpu.async_copy(rows.at[0], acc.at[didx.at[c]],
                                         dsems[b], add=True)
                return 0
            lax.fori_loop(0, (n_ch + nd - 1) // nd - 1, _grp, 0)
            rem = n_ch - ((n_ch + nd - 1) // nd - 1) * nd
            for b in range(rem):
                pltpu.make_async_copy(rows.at[0], acc.at[didx.at[0]],
                                      dsems[b]).wait()
        plsc.subcore_barrier()

        out0 = pl.multiple_of(cid * n + row0, 8)
        pltpu.sync_copy(acc.at[pl.ds(row0, rpt)], out_hbm.at[pl.ds(out0, rpt)])

        @pl.when(sid == _NS - 1)
        def _():
            pltpu.sync_copy(acc.at[pl.ds(_NS * rpt, last_extra)],
                            out_hbm.at[pl.ds(cid * n + _NS * rpt, last_extra)])

    return seg


def _dense_layer(n, d, h, br):
    grid = (n // br,)

    def body(x_ref, p_ref, dg_ref, ws_ref, wn_ref, b_ref, o_ref):
        p = p_ref[0] + p_ref[1]
        deg = dg_ref[0] + dg_ref[1]
        hn = p / jnp.maximum(deg, 1.0)
        y = (jnp.dot(x_ref[...], ws_ref[...], preferred_element_type=jnp.float32)
             + jnp.dot(hn, wn_ref[...], preferred_element_type=jnp.float32)
             + b_ref[...])
        o_ref[...] = jnp.maximum(y, 0.0)

    return pl.pallas_call(
        body,
        grid=grid,
        in_specs=[
            pl.BlockSpec((br, d), lambda i: (i, 0)),
            pl.BlockSpec((_NC, br, d), lambda i: (0, i, 0)),
            pl.BlockSpec((_NC, br, d), lambda i: (0, i, 0)),
            pl.BlockSpec((d, h), lambda i: (0, 0)),
            pl.BlockSpec((d, h), lambda i: (0, 0)),
            pl.BlockSpec((1, h), lambda i: (0, 0)),
        ],
        out_specs=pl.BlockSpec((br, h), lambda i: (i, 0)),
        out_shape=jax.ShapeDtypeStruct((n, h), jnp.float32),
    )


def _final_layer(n, d, h, fc2, c, br):
    nb = n // br

    def body(x_ref, p_ref, dg_ref, ws_ref, wn_ref, b_ref,
             wf1_ref, bf1_ref, wf2_ref, bf2_ref, o_ref, acc_ref):
        i = pl.program_id(0)

        @pl.when(i == 0)
        def _():
            acc_ref[...] = jnp.zeros_like(acc_ref)

        p = p_ref[0] + p_ref[1]
        deg = dg_ref[0] + dg_ref[1]
        hn = p / jnp.maximum(deg, 1.0)
        y = (jnp.dot(x_ref[...], ws_ref[...], preferred_element_type=jnp.float32)
             + jnp.dot(hn, wn_ref[...], preferred_element_type=jnp.float32)
             + b_ref[...])
        y = jnp.maximum(y, 0.0)
        acc_ref[...] += jnp.sum(y, axis=0, keepdims=True)

        @pl.when(i == nb - 1)
        def _():
            hg = acc_ref[...] / float(n)
            t1 = jnp.dot(hg, wf1_ref[...], preferred_element_type=jnp.float32)
            t1 = jnp.maximum(t1 + bf1_ref[...], 0.0)
            t2 = jnp.dot(t1, wf2_ref[...], preferred_element_type=jnp.float32)
            t2 = t2 + bf2_ref[...]
            m = jnp.max(t2)
            lse = m + jnp.log(jnp.sum(jnp.exp(t2 - m)))
            o_ref[...] = t2 - lse

    return pl.pallas_call(
        body,
        grid=(nb,),
        in_specs=[
            pl.BlockSpec((br, d), lambda i: (i, 0)),
            pl.BlockSpec((_NC, br, d), lambda i: (0, i, 0)),
            pl.BlockSpec((_NC, br, d), lambda i: (0, i, 0)),
            pl.BlockSpec((d, h), lambda i: (0, 0)),
            pl.BlockSpec((d, h), lambda i: (0, 0)),
            pl.BlockSpec((1, h), lambda i: (0, 0)),
            pl.BlockSpec((h, fc2), lambda i: (0, 0)),
            pl.BlockSpec((1, fc2), lambda i: (0, 0)),
            pl.BlockSpec((fc2, c), lambda i: (0, 0)),
            pl.BlockSpec((1, c), lambda i: (0, 0)),
        ],
        out_specs=pl.BlockSpec((1, c), lambda i: (0, 0)),
        out_shape=jax.ShapeDtypeStruct((1, c), jnp.float32),
        scratch_shapes=[pltpu.VMEM((1, h), jnp.float32)],
    )


def kernel(x, edge_index, W1_self, W1_neigh, b1, W2_self, W2_neigh, b2,
           W3_self, W3_neigh, b3, Wfc1, bfc1, Wfc2, bfc2):
    n, d = x.shape
    e = edge_index.shape[1]
    h1 = W1_self.shape[1]
    h2 = W2_self.shape[1]
    fc1 = W3_self.shape[1]
    fc2 = Wfc1.shape[1]
    c = Wfc2.shape[1]
    br = 2000

    src3 = edge_index[0]
    dst3 = edge_index[1]

    seg = _seg_sum_kernel(n, d, e, with_gather=True)
    degk = _seg_sum_kernel(n, d, e, with_gather=False)
    dense1 = _dense_layer(n, d, h1, br)
    dense2 = _dense_layer(n, h1, h2, br)
    dense3 = _final_layer(n, h2, fc1, fc2, c, br)

    degw = degk(x, src3, dst3).reshape(_NC, n, d)

    p1 = seg(x, src3, dst3).reshape(_NC, n, d)
    hh1 = dense1(x, p1, degw, W1_self, W1_neigh, b1.reshape(1, h1))
    p2 = seg(hh1, src3, dst3).reshape(_NC, n, d)
    hh2 = dense2(hh1, p2, degw, W2_self, W2_neigh, b2.reshape(1, h2))
    p3 = seg(hh2, src3, dst3).reshape(_NC, n, d)
    out = dense3(hh2, p3, degw, W3_self, W3_neigh, b3.reshape(1, fc1),
                 Wfc1, bfc1.reshape(1, fc2), Wfc2, bfc2.reshape(1, c))
    return out


# speedup vs baseline: 9.5599x; 1.1507x over previous
import functools

import jax
import jax.numpy as jnp
from jax import lax
from jax.experimental import pallas as pl
from jax.experimental.pallas import tpu as pltpu
from jax.experimental.pallas import tpu_sc as plsc

_NC, _NS, _L = 2, 16, 16
_NW = _NC * _NS


def _seg_sum_kernel(n, d, e, with_gather):
    per_w = e // _NW
    ch = 80
    n_ch = per_w // ch
    NB = 2
    n_grp = n_ch // NB
    n_tail = n_ch - n_grp * NB
    rpt = (n // _NS) // 8 * 8
    last_extra = n - _NS * rpt
    nz = rpt // ch
    zrem = rpt - nz * ch
    mesh = plsc.VectorSubcoreMesh(core_axis_name="c", subcore_axis_name="s")

    scratch = [
        pltpu.VMEM((NB, ch), jnp.int32),
        pltpu.VMEM((n_ch, ch), jnp.int32),
        pltpu.VMEM((NB, ch, d), jnp.float32),
        pltpu.VMEM_SHARED((n, d), jnp.float32),
    ] + [pltpu.SemaphoreType.DMA] * 8

    @functools.partial(
        pl.kernel,
        out_type=jax.ShapeDtypeStruct((_NC * n, d), jnp.float32),
        mesh=mesh,
        scratch_types=scratch,
    )
    def seg(x_hbm, src_hbm, dst_hbm, out_hbm, sidx, didx, rows, acc,
            *sems):
        gsem, isem, ssem = sems[:NB], sems[NB:2 * NB], sems[2 * NB:3 * NB]
        cid = lax.axis_index("c")
        sid = lax.axis_index("s")
        wid = cid * _NS + sid
        base = wid * per_w

        def _wait_idx(sem):
            pltpu.make_async_copy(dst_hbm.at[pl.ds(0, ch)], sidx.at[0],
                                  sem).wait()

        def _wait_gather(b):
            pltpu.make_async_copy(x_hbm.at[sidx.at[b]], rows.at[b],
                                  gsem[b]).wait()

        ni = len(sems)
        ip_tail = n_ch % ni

        def _idx(t, _):
            for b in range(ni):
                k = t * ni + b

                @pl.when(t > 0)
                def _():
                    _wait_idx(sems[b])
                pltpu.async_copy(dst_hbm.at[pl.ds(base + k * ch, ch)],
                                 didx.at[k], sems[b])
            return 0
        lax.fori_loop(0, n_ch // ni, _idx, 0)
        for r in range(ip_tail):
            k = (n_ch // ni) * ni + r
            _wait_idx(sems[r])
            pltpu.async_copy(dst_hbm.at[pl.ds(base + k * ch, ch)],
                             didx.at[k], sems[r])

        def _zr(r, _):
            def _zc(c, _):
                rows[0, r, pl.ds(c * _L, _L)] = jnp.zeros((_L,), jnp.float32)
                return 0
            return lax.fori_loop(0, d // _L, _zc, 0)
        lax.fori_loop(0, ch, _zr, 0)

        for b in range(ni):
            _wait_idx(sems[b])

        row0 = pl.multiple_of(sid * rpt, 8)
        for j in range(nz):
            pltpu.sync_copy(rows.at[0], acc.at[pl.ds(row0 + j * ch, ch)])
        pltpu.sync_copy(rows.at[0, pl.ds(0, zrem)],
                        acc.at[pl.ds(row0 + nz * ch, zrem)])

        @pl.when(sid == _NS - 1)
        def _():
            pltpu.sync_copy(rows.at[0, pl.ds(0, last_extra)],
                            acc.at[pl.ds(_NS * rpt, last_extra)])

        if with_gather:
            for b in range(NB):
                pltpu.async_copy(src_hbm.at[pl.ds(base + b * ch, ch)],
                                 sidx.at[b], isem[b])
            _wait_idx(isem[0])
            pltpu.async_copy(x_hbm.at[sidx.at[0]], rows.at[0], gsem[0])
        else:
            def _or(r, _):
                def _oc(c, _):
                    rows[0, r, pl.ds(c * _L, _L)] = jnp.full((_L,), 1.0,
                                                             jnp.float32)
                    return 0
                return lax.fori_loop(0, d // _L, _oc, 0)
            lax.fori_loop(0, ch, _or, 0)
        plsc.subcore_barrier()

        if with_gather:
            def _wait_scat(b):
                pltpu.make_async_copy(rows.at[b], acc.at[didx.at[0]],
                                      ssem[b]).wait()

            def _step(c, b, first):
                nb = 1 - b

                @pl.when(c + 1 < n_ch)
                def _():
                    _wait_idx(isem[nb])
                    if not first:
                        _wait_scat(nb)
                    pltpu.async_copy(x_hbm.at[sidx.at[nb]], rows.at[nb],
                                     gsem[nb])
                _wait_gather(b)
                pltpu.async_copy(rows.at[b], acc.at[didx.at[c]], ssem[b],
                                 add=True)

                @pl.when(c + 2 < n_ch)
                def _():
                    pltpu.async_copy(
                        src_hbm.at[pl.ds(base + (c + 2) * ch, ch)],
                        sidx.at[b], isem[b])

            _step(0, 0, True)

            def _grp(t, _):
                for b in range(NB):
                    _step(1 + t * NB + b, 1 - b if NB == 2 else b, False)
                return 0
            lax.fori_loop(0, (n_ch - 1) // NB, _grp, 0)
            _wait_scat((n_ch - 2) % NB)
            _wait_scat((n_ch - 1) % NB)
        else:
            dsems = list(sems)
            nd = len(dsems)
            for b in range(nd):
                pltpu.async_copy(rows.at[0], acc.at[didx.at[b]], dsems[b],
                                 add=True)

            def _grp(t, _):
                for b in range(nd):
                    c = (t + 1) * nd + b
                    pltpu.make_async_copy(rows.at[0], acc.at[didx.at[0]],
                                          dsems[b]).wait()

                    @pl.when(c < n_ch)
                    def _():
                        pltpu.async_copy(rows.at[0], acc.at[didx.at[c]],
                                         dsems[b], add=True)
                return 0
            lax.fori_loop(0, (n_ch + nd - 1) // nd - 1, _grp, 0)
            rem = n_ch - ((n_ch + nd - 1) // nd - 1) * nd
            for b in range(rem):
                pltpu.make_async_copy(rows.at[0], acc.at[didx.at[0]],
                                      dsems[b]).wait()
        plsc.subcore_barrier()

        out0 = pl.multiple_of(cid * n + row0, 8)
        pltpu.sync_copy(acc.at[pl.ds(row0, rpt)], out_hbm.at[pl.ds(out0, rpt)])

        @pl.when(sid == _NS - 1)
        def _():
            pltpu.sync_copy(acc.at[pl.ds(_NS * rpt, last_extra)],
                            out_hbm.at[pl.ds(cid * n + _NS * rpt, last_extra)])

    return seg


def _dense_layer(n, d, h, br):
    grid = (n // br,)

    def body(x_ref, p_ref, dg_ref, ws_ref, wn_ref, b_ref, o_ref):
        p = p_ref[0] + p_ref[1]
        deg = dg_ref[0] + dg_ref[1]
        hn = p / jnp.maximum(deg, 1.0)
        y = (jnp.dot(x_ref[...], ws_ref[...], preferred_element_type=jnp.float32)
             + jnp.dot(hn, wn_ref[...], preferred_element_type=jnp.float32)
             + b_ref[...])
        o_ref[...] = jnp.maximum(y, 0.0)

    return pl.pallas_call(
        body,
        grid=grid,
        in_specs=[
            pl.BlockSpec((br, d), lambda i: (i, 0)),
            pl.BlockSpec((_NC, br, d), lambda i: (0, i, 0)),
            pl.BlockSpec((_NC, br, d), lambda i: (0, i, 0)),
            pl.BlockSpec((d, h), lambda i: (0, 0)),
            pl.BlockSpec((d, h), lambda i: (0, 0)),
            pl.BlockSpec((1, h), lambda i: (0, 0)),
        ],
        out_specs=pl.BlockSpec((br, h), lambda i: (i, 0)),
        out_shape=jax.ShapeDtypeStruct((n, h), jnp.float32),
    )


def _final_layer(n, d, h, fc2, c, br):
    nb = n // br

    def body(x_ref, p_ref, dg_ref, ws_ref, wn_ref, b_ref,
             wf1_ref, bf1_ref, wf2_ref, bf2_ref, o_ref, acc_ref):
        i = pl.program_id(0)

        @pl.when(i == 0)
        def _():
            acc_ref[...] = jnp.zeros_like(acc_ref)

        p = p_ref[0] + p_ref[1]
        deg = dg_ref[0] + dg_ref[1]
        hn = p / jnp.maximum(deg, 1.0)
        y = (jnp.dot(x_ref[...], ws_ref[...], preferred_element_type=jnp.float32)
             + jnp.dot(hn, wn_ref[...], preferred_element_type=jnp.float32)
             + b_ref[...])
        y = jnp.maximum(y, 0.0)
        acc_ref[...] += jnp.sum(y, axis=0, keepdims=True)

        @pl.when(i == nb - 1)
        def _():
            hg = acc_ref[...] / float(n)
            t1 = jnp.dot(hg, wf1_ref[...], preferred_element_type=jnp.float32)
            t1 = jnp.maximum(t1 + bf1_ref[...], 0.0)
            t2 = jnp.dot(t1, wf2_ref[...], preferred_element_type=jnp.float32)
            t2 = t2 + bf2_ref[...]
            m = jnp.max(t2)
            lse = m + jnp.log(jnp.sum(jnp.exp(t2 - m)))
            o_ref[...] = t2 - lse

    return pl.pallas_call(
        body,
        grid=(nb,),
        in_specs=[
            pl.BlockSpec((br, d), lambda i: (i, 0)),
            pl.BlockSpec((_NC, br, d), lambda i: (0, i, 0)),
            pl.BlockSpec((_NC, br, d), lambda i: (0, i, 0)),
            pl.BlockSpec((d, h), lambda i: (0, 0)),
            pl.BlockSpec((d, h), lambda i: (0, 0)),
            pl.BlockSpec((1, h), lambda i: (0, 0)),
            pl.BlockSpec((h, fc2), lambda i: (0, 0)),
            pl.BlockSpec((1, fc2), lambda i: (0, 0)),
            pl.BlockSpec((fc2, c), lambda i: (0, 0)),
            pl.BlockSpec((1, c), lambda i: (0, 0)),
        ],
        out_specs=pl.BlockSpec((1, c), lambda i: (0, 0)),
        out_shape=jax.ShapeDtypeStruct((1, c), jnp.float32),
        scratch_shapes=[pltpu.VMEM((1, h), jnp.float32)],
    )


def kernel(x, edge_index, W1_self, W1_neigh, b1, W2_self, W2_neigh, b2,
           W3_self, W3_neigh, b3, Wfc1, bfc1, Wfc2, bfc2):
    n, d = x.shape
    e = edge_index.shape[1]
    h1 = W1_self.shape[1]
    h2 = W2_self.shape[1]
    fc1 = W3_self.shape[1]
    fc2 = Wfc1.shape[1]
    c = Wfc2.shape[1]
    br = 2000

    src3 = edge_index[0]
    dst3 = edge_index[1]

    seg = _seg_sum_kernel(n, d, e, with_gather=True)
    degk = _seg_sum_kernel(n, d, e, with_gather=False)
    dense1 = _dense_layer(n, d, h1, br)
    dense2 = _dense_layer(n, h1, h2, br)
    dense3 = _final_layer(n, h2, fc1, fc2, c, br)

    degw = degk(x, src3, dst3).reshape(_NC, n, d)

    p1 = seg(x, src3, dst3).reshape(_NC, n, d)
    hh1 = dense1(x, p1, degw, W1_self, W1_neigh, b1.reshape(1, h1))
    p2 = seg(hh1, src3, dst3).reshape(_NC, n, d)
    hh2 = dense2(hh1, p2, degw, W2_self, W2_neigh, b2.reshape(1, h2))
    p3 = seg(hh2, src3, dst3).reshape(_NC, n, d)
    out = dense3(hh2, p3, degw, W3_self, W3_neigh, b3.reshape(1, fc1),
                 Wfc1, bfc1.reshape(1, fc2), Wfc2, bfc2.reshape(1, c))
    return out
